# baseline (speedup 1.0000x reference)
import functools

import jax
import jax.numpy as jnp
from jax import lax
from jax.experimental import pallas as pl
from jax.experimental.pallas import tpu as pltpu
from jax.experimental.pallas import tpu_sc as plsc

N = 10000
E = 320000
D = 128
SIGMA = 0.1

NC = 2
NS = 16
NW = NC * NS
CHUNK = 96
NRING = 3
IRING = 4
UNROLL = 12
NCHUNK = -(-E // (NW * CHUNK * UNROLL)) * UNROLL
EPAD = NW * CHUNK * NCHUNK
NPAD = ((N + NS * 8) // (NS * 8)) * (NS * 8)
ROWS_PER_TILE = NPAD // NS


def _normalize_body(x_ref, o_ref):
    x = x_ref[...]
    s = jnp.sum(x * x, axis=1, keepdims=True)
    o_ref[...] = x * lax.rsqrt(jnp.maximum(s, 1e-24))


def _normalize(x):
    return pl.pallas_call(
        _normalize_body,
        out_shape=jax.ShapeDtypeStruct((N, D), jnp.float32),
    )(x)


_sc_mesh = plsc.VectorSubcoreMesh(core_axis_name="c", subcore_axis_name="s")


@functools.partial(
    pl.kernel,
    out_type=(
        jax.ShapeDtypeStruct((NC, NPAD, D), jnp.float32),
        jax.ShapeDtypeStruct((NW, NPAD), jnp.float32),
    ),
    mesh=_sc_mesh,
    scratch_types=[
        [pltpu.VMEM((2, CHUNK), jnp.int32)] * IRING,
        [pltpu.VMEM((CHUNK, D), jnp.float32)] * NRING,
        pltpu.VMEM((NPAD,), jnp.float32),
        pltpu.VMEM_SHARED((NPAD, D), jnp.float32),
        [pltpu.SemaphoreType.DMA] * IRING,
        [pltpu.SemaphoreType.DMA] * NRING,
    ],
    compiler_params=pltpu.CompilerParams(needs_layout_passes=False),
)
def _sc_scatter(xn_hbm, idx_hbm, zrow_hbm, zdeg_hbm,
                agg_out, deg_out, ib, rows, deg_v, agg_sh, isems, gsems):
    c = lax.axis_index("c")
    s = lax.axis_index("s")
    g = c * NS + s

    pltpu.sync_copy(zrow_hbm, agg_sh.at[pl.ds(s * ROWS_PER_TILE, ROWS_PER_TILE)])
    pltpu.sync_copy(zdeg_hbm, deg_v)
    plsc.subcore_barrier()

    ones = jnp.full((16,), 1.0, dtype=jnp.float32)

    def idx_fetch(j, a):
        pltpu.async_copy(idx_hbm.at[g, j], ib[a], isems[a])

    def idx_wait(j, a):
        pltpu.make_async_copy(idx_hbm.at[g, j], ib[a], isems[a]).wait()

    def row_gather(a, b):
        pltpu.async_copy(xn_hbm.at[ib[a].at[0]], rows[b], gsems[b])

    def row_wait(a, b):
        pltpu.make_async_copy(xn_hbm.at[ib[a].at[0]], rows[b], gsems[b]).wait()

    for a in range(IRING):
        idx_fetch(a, a)
    for a in range(2):
        idx_wait(a, a)
        row_gather(a, a)

    def body(jg, carry):
        for u in range(UNROLL):
            j0 = jg * UNROLL + u
            b = u % NRING
            a = u % IRING

            @pl.when(j0 + 2 < NCHUNK)
            def _():
                a2 = (u + 2) % IRING
                idx_wait(j0 + 2, a2)
                row_gather(a2, (u + 2) % NRING)

            row_wait(a, b)
            pltpu.sync_copy(rows[b], agg_sh.at[ib[a].at[1]], add=True)
            for i in range(CHUNK // 16):
                idx16 = ib[a][1, pl.ds(i * 16, 16)]
                plsc.addupdate_scatter(deg_v, [idx16], ones)

            @pl.when(j0 + IRING < NCHUNK)
            def _():
                idx_fetch(j0 + IRING, a)

        return carry

    lax.fori_loop(0, NCHUNK // UNROLL, body, 0)
    plsc.subcore_barrier()

    pltpu.sync_copy(
        agg_sh.at[pl.ds(s * ROWS_PER_TILE, ROWS_PER_TILE)],
        agg_out.at[c, pl.ds(s * ROWS_PER_TILE, ROWS_PER_TILE)],
    )
    pltpu.sync_copy(deg_v, deg_out.at[g])


_FIN_BLK = 1000


def _final_body(a0_ref, a1_ref, deg_ref, w_ref, b_ref, noise_ref, o_ref):
    deg = jnp.sum(deg_ref[..., 0], axis=0)
    agg = a0_ref[0] + a1_ref[0]
    mean = agg / jnp.maximum(deg, 1.0)[:, None]
    o_ref[...] = (
        jnp.dot(mean, w_ref[...], preferred_element_type=jnp.float32)
        + b_ref[...]
        + noise_ref[...] * SIGMA
    )


def _final(agg_part, deg_part3, W, b2, noise):
    return pl.pallas_call(
        _final_body,
        grid=(N // _FIN_BLK,),
        in_specs=[
            pl.BlockSpec((1, _FIN_BLK, D), lambda i: (0, i, 0)),
            pl.BlockSpec((1, _FIN_BLK, D), lambda i: (1, i, 0)),
            pl.BlockSpec((NW, _FIN_BLK, 1), lambda i: (0, i, 0)),
            pl.BlockSpec((D, D), lambda i: (0, 0)),
            pl.BlockSpec((1, D), lambda i: (0, 0)),
            pl.BlockSpec((_FIN_BLK, D), lambda i: (i, 0)),
        ],
        out_specs=pl.BlockSpec((_FIN_BLK, D), lambda i: (i, 0)),
        out_shape=jax.ShapeDtypeStruct((N, D), jnp.float32),
    )(agg_part, agg_part, deg_part3, W, b2, noise)


def kernel(x, edge_index, W, b, noise):
    xn = _normalize(x)

    src = edge_index[0]
    dst = edge_index[1]
    pad = EPAD - E
    src_p = jnp.concatenate([src, jnp.zeros((pad,), jnp.int32)])
    dst_p = jnp.concatenate([dst, jnp.full((pad,), N, jnp.int32)])
    idx4 = jnp.stack(
        [src_p.reshape(NW, NCHUNK, CHUNK), dst_p.reshape(NW, NCHUNK, CHUNK)],
        axis=2,
    )

    zrow = jnp.zeros((ROWS_PER_TILE, D), jnp.float32)
    zdeg = jnp.zeros((NPAD,), jnp.float32)

    agg_part, deg_part = _sc_scatter(xn, idx4, zrow, zdeg)

    return _final(agg_part, deg_part.reshape(NW, NPAD, 1), W,
                  b.reshape(1, D), noise)


# speedup vs baseline: 1.8126x; 1.8126x over previous
import functools

import jax
import jax.numpy as jnp
from jax import lax
from jax.experimental import pallas as pl
from jax.experimental.pallas import tpu as pltpu
from jax.experimental.pallas import tpu_sc as plsc

N = 10000
E = 320000
D = 128
SIGMA = 0.1

NC = 2
NS = 16
NW = NC * NS
CHUNK = 96
NRING = 3
IRING = 3
UNROLL = 3
NCHUNK = -(-E // (NW * CHUNK * UNROLL)) * UNROLL
EPAD = NW * CHUNK * NCHUNK
NPAD = ((N + NS * 8) // (NS * 8)) * (NS * 8)
ROWS_PER_TILE = NPAD // NS


def _normalize_body(x_ref, o_ref):
    x = x_ref[...]
    s = jnp.sum(x * x, axis=1, keepdims=True)
    o_ref[...] = x * lax.rsqrt(jnp.maximum(s, 1e-24))


def _normalize(x):
    return pl.pallas_call(
        _normalize_body,
        out_shape=jax.ShapeDtypeStruct((N, D), jnp.float32),
    )(x)


_sc_mesh = plsc.VectorSubcoreMesh(core_axis_name="c", subcore_axis_name="s")


@functools.partial(
    pl.kernel,
    out_type=(
        jax.ShapeDtypeStruct((NC, NPAD, D), jnp.float32),
        jax.ShapeDtypeStruct((NW, NPAD), jnp.float32),
    ),
    mesh=_sc_mesh,
    scratch_types=[
        [pltpu.VMEM((2, CHUNK), jnp.int32)] * IRING,
        [pltpu.VMEM((CHUNK, D), jnp.float32)] * NRING,
        pltpu.VMEM((NPAD,), jnp.float32),
        pltpu.VMEM_SHARED((NPAD, D), jnp.float32),
        [pltpu.SemaphoreType.DMA] * IRING,
        [pltpu.SemaphoreType.DMA] * NRING,
    ],
    compiler_params=pltpu.CompilerParams(needs_layout_passes=False),
)
def _sc_scatter(xn_hbm, idx_hbm, zrow_hbm, zdeg_hbm,
                agg_out, deg_out, ib, rows, deg_v, agg_sh, isems, gsems):
    c = lax.axis_index("c")
    s = lax.axis_index("s")
    g = c * NS + s

    pltpu.sync_copy(zrow_hbm, agg_sh.at[pl.ds(s * ROWS_PER_TILE, ROWS_PER_TILE)])
    pltpu.sync_copy(zdeg_hbm, deg_v)
    plsc.subcore_barrier()

    ones = jnp.full((16,), 1.0, dtype=jnp.float32)

    def idx_fetch(j, a):
        pltpu.async_copy(idx_hbm.at[g, j], ib[a], isems[a])

    def idx_wait(j, a):
        pltpu.make_async_copy(idx_hbm.at[g, j], ib[a], isems[a]).wait()

    def row_gather(a, b):
        pltpu.async_copy(xn_hbm.at[ib[a].at[0]], rows[b], gsems[b])

    def row_wait(a, b):
        pltpu.make_async_copy(xn_hbm.at[ib[a].at[0]], rows[b], gsems[b]).wait()

    for a in range(IRING):
        idx_fetch(a, a)
    for a in range(2):
        idx_wait(a, a)
        row_gather(a, a)

    def body(jg, carry):
        for u in range(UNROLL):
            j0 = jg * UNROLL + u
            b = u % NRING
            a = u % IRING

            @pl.when(j0 + 2 < NCHUNK)
            def _():
                a2 = (u + 2) % IRING
                idx_wait(j0 + 2, a2)
                row_gather(a2, (u + 2) % NRING)

            row_wait(a, b)
            pltpu.sync_copy(rows[b], agg_sh.at[ib[a].at[1]], add=True)
            for i in range(CHUNK // 16):
                idx16 = ib[a][1, pl.ds(i * 16, 16)]
                plsc.addupdate_scatter(deg_v, [idx16], ones)

            @pl.when(j0 + IRING < NCHUNK)
            def _():
                idx_fetch(j0 + IRING, a)

        return carry

    lax.fori_loop(0, NCHUNK // UNROLL, body, 0)
    plsc.subcore_barrier()

    pltpu.sync_copy(
        agg_sh.at[pl.ds(s * ROWS_PER_TILE, ROWS_PER_TILE)],
        agg_out.at[c, pl.ds(s * ROWS_PER_TILE, ROWS_PER_TILE)],
    )
    pltpu.sync_copy(deg_v, deg_out.at[g])


_FIN_BLK = 1000


def _final_body(a0_ref, a1_ref, deg_ref, w_ref, b_ref, noise_ref, o_ref):
    deg = jnp.sum(deg_ref[..., 0], axis=0)
    agg = a0_ref[0] + a1_ref[0]
    mean = agg / jnp.maximum(deg, 1.0)[:, None]
    o_ref[...] = (
        jnp.dot(mean, w_ref[...], preferred_element_type=jnp.float32)
        + b_ref[...]
        + noise_ref[...] * SIGMA
    )


def _final(agg_part, deg_part3, W, b2, noise):
    return pl.pallas_call(
        _final_body,
        grid=(N // _FIN_BLK,),
        in_specs=[
            pl.BlockSpec((1, _FIN_BLK, D), lambda i: (0, i, 0)),
            pl.BlockSpec((1, _FIN_BLK, D), lambda i: (1, i, 0)),
            pl.BlockSpec((NW, _FIN_BLK, 1), lambda i: (0, i, 0)),
            pl.BlockSpec((D, D), lambda i: (0, 0)),
            pl.BlockSpec((1, D), lambda i: (0, 0)),
            pl.BlockSpec((_FIN_BLK, D), lambda i: (i, 0)),
        ],
        out_specs=pl.BlockSpec((_FIN_BLK, D), lambda i: (i, 0)),
        out_shape=jax.ShapeDtypeStruct((N, D), jnp.float32),
    )(agg_part, agg_part, deg_part3, W, b2, noise)


def kernel(x, edge_index, W, b, noise):
    xn = _normalize(x)

    src = edge_index[0]
    dst = edge_index[1]
    pad = EPAD - E
    src_p = jnp.concatenate([src, jnp.zeros((pad,), jnp.int32)])
    dst_p = jnp.concatenate([dst, jnp.full((pad,), N, jnp.int32)])
    idx4 = jnp.stack(
        [src_p.reshape(NW, NCHUNK, CHUNK), dst_p.reshape(NW, NCHUNK, CHUNK)],
        axis=2,
    )

    zrow = jnp.zeros((ROWS_PER_TILE, D), jnp.float32)
    zdeg = jnp.zeros((NPAD,), jnp.float32)

    agg_part, deg_part = _sc_scatter(xn, idx4, zrow, zdeg)

    return _final(agg_part, deg_part.reshape(NW, NPAD, 1), W,
                  b.reshape(1, D), noise)


# speedup vs baseline: 1.8748x; 1.0343x over previous
import functools

import jax
import jax.numpy as jnp
from jax import lax
from jax.experimental import pallas as pl
from jax.experimental.pallas import tpu as pltpu
from jax.experimental.pallas import tpu_sc as plsc

N = 10000
E = 320000
D = 128
SIGMA = 0.1

NC = 2
NS = 16
NW = NC * NS
CHUNK = 96
NRING = 3
IRING = 3
UNROLL = 3
NCHUNK = -(-E // (NW * CHUNK * UNROLL)) * UNROLL
EPAD = NW * CHUNK * NCHUNK
NPAD = ((N + NS * 8) // (NS * 8)) * (NS * 8)
ROWS_PER_TILE = NPAD // NS


def _normalize_body(x_ref, o_ref):
    x = x_ref[...]
    s = jnp.sum(x * x, axis=1, keepdims=True)
    o_ref[...] = x * lax.rsqrt(jnp.maximum(s, 1e-24))


def _normalize(x):
    return pl.pallas_call(
        _normalize_body,
        out_shape=jax.ShapeDtypeStruct((N, D), jnp.float32),
    )(x)


_sc_mesh = plsc.VectorSubcoreMesh(core_axis_name="c", subcore_axis_name="s")


@functools.partial(
    pl.kernel,
    out_type=(
        jax.ShapeDtypeStruct((NC, NPAD, D), jnp.float32),
        jax.ShapeDtypeStruct((NW, NPAD), jnp.float32),
    ),
    mesh=_sc_mesh,
    scratch_types=[
        [pltpu.VMEM((2, CHUNK), jnp.int32)] * IRING,
        [pltpu.VMEM((CHUNK, D), jnp.float32)] * NRING,
        pltpu.VMEM((NPAD,), jnp.float32),
        pltpu.VMEM_SHARED((NPAD, D), jnp.float32),
        [pltpu.SemaphoreType.DMA] * IRING,
        [pltpu.SemaphoreType.DMA] * NRING,
    ],
    compiler_params=pltpu.CompilerParams(needs_layout_passes=False),
)
def _sc_scatter(xn_hbm, idx_hbm, zrow_hbm, zdeg_hbm,
                agg_out, deg_out, ib, rows, deg_v, agg_sh, isems, gsems):
    c = lax.axis_index("c")
    s = lax.axis_index("s")
    g = c * NS + s

    pltpu.sync_copy(zrow_hbm, agg_sh.at[pl.ds(s * ROWS_PER_TILE, ROWS_PER_TILE)])
    pltpu.sync_copy(zdeg_hbm, deg_v)
    plsc.subcore_barrier()

    ones = jnp.full((16,), 1.0, dtype=jnp.float32)

    def idx_fetch(j, a):
        pltpu.async_copy(idx_hbm.at[g, j], ib[a], isems[a])

    def idx_wait(j, a):
        pltpu.make_async_copy(idx_hbm.at[g, j], ib[a], isems[a]).wait()

    def row_gather(a, b):
        pltpu.async_copy(xn_hbm.at[ib[a].at[0]], rows[b], gsems[b])

    def row_wait(a, b):
        pltpu.make_async_copy(xn_hbm.at[ib[a].at[0]], rows[b], gsems[b]).wait()

    for a in range(IRING):
        idx_fetch(a, a)
    for a in range(2):
        idx_wait(a, a)
        row_gather(a, a)

    def body(jg, carry):
        for u in range(UNROLL):
            j0 = jg * UNROLL + u
            b = u % NRING
            a = u % IRING

            row_wait(a, b)
            pltpu.sync_copy(rows[b], agg_sh.at[ib[a].at[1]], add=True)
            for i in range(CHUNK // 16):
                idx16 = ib[a][1, pl.ds(i * 16, 16)]
                plsc.addupdate_scatter(deg_v, [idx16], ones)

            @pl.when(j0 + IRING < NCHUNK)
            def _():
                idx_fetch(j0 + IRING, a)

            @pl.when(j0 + 2 < NCHUNK)
            def _():
                a2 = (u + 2) % IRING
                idx_wait(j0 + 2, a2)
                row_gather(a2, (u + 2) % NRING)

        return carry

    lax.fori_loop(0, NCHUNK // UNROLL, body, 0)
    plsc.subcore_barrier()

    pltpu.sync_copy(
        agg_sh.at[pl.ds(s * ROWS_PER_TILE, ROWS_PER_TILE)],
        agg_out.at[c, pl.ds(s * ROWS_PER_TILE, ROWS_PER_TILE)],
    )
    pltpu.sync_copy(deg_v, deg_out.at[g])


_FIN_BLK = 1000


def _final_body(a0_ref, a1_ref, deg_ref, w_ref, b_ref, noise_ref, o_ref):
    deg = jnp.sum(deg_ref[..., 0], axis=0)
    agg = a0_ref[0] + a1_ref[0]
    mean = agg / jnp.maximum(deg, 1.0)[:, None]
    o_ref[...] = (
        jnp.dot(mean, w_ref[...], preferred_element_type=jnp.float32)
        + b_ref[...]
        + noise_ref[...] * SIGMA
    )


def _final(agg_part, deg_part3, W, b2, noise):
    return pl.pallas_call(
        _final_body,
        grid=(N // _FIN_BLK,),
        in_specs=[
            pl.BlockSpec((1, _FIN_BLK, D), lambda i: (0, i, 0)),
            pl.BlockSpec((1, _FIN_BLK, D), lambda i: (1, i, 0)),
            pl.BlockSpec((NW, _FIN_BLK, 1), lambda i: (0, i, 0)),
            pl.BlockSpec((D, D), lambda i: (0, 0)),
            pl.BlockSpec((1, D), lambda i: (0, 0)),
            pl.BlockSpec((_FIN_BLK, D), lambda i: (i, 0)),
        ],
        out_specs=pl.BlockSpec((_FIN_BLK, D), lambda i: (i, 0)),
        out_shape=jax.ShapeDtypeStruct((N, D), jnp.float32),
    )(agg_part, agg_part, deg_part3, W, b2, noise)


def kernel(x, edge_index, W, b, noise):
    xn = _normalize(x)

    src = edge_index[0]
    dst = edge_index[1]
    pad = EPAD - E
    src_p = jnp.concatenate([src, jnp.zeros((pad,), jnp.int32)])
    dst_p = jnp.concatenate([dst, jnp.full((pad,), N, jnp.int32)])
    idx4 = jnp.stack(
        [src_p.reshape(NW, NCHUNK, CHUNK), dst_p.reshape(NW, NCHUNK, CHUNK)],
        axis=2,
    )

    zrow = jnp.zeros((ROWS_PER_TILE, D), jnp.float32)
    zdeg = jnp.zeros((NPAD,), jnp.float32)

    agg_part, deg_part = _sc_scatter(xn, idx4, zrow, zdeg)

    return _final(agg_part, deg_part.reshape(NW, NPAD, 1), W,
                  b.reshape(1, D), noise)


# speedup vs baseline: 2.7242x; 1.4530x over previous
import functools

import jax
import jax.numpy as jnp
from jax import lax
from jax.experimental import pallas as pl
from jax.experimental.pallas import tpu as pltpu
from jax.experimental.pallas import tpu_sc as plsc

N = 10000
E = 320000
D = 128
SIGMA = 0.1

NC = 2
NS = 16
NW = NC * NS
CHUNK = 96
NRING = 3
IRING = 3
UNROLL = 3
NCHUNK = -(-E // (NW * CHUNK * UNROLL)) * UNROLL
EPAD = NW * CHUNK * NCHUNK
NPAD = 10240
ROWS_PER_TILE = NPAD // NS


def _normalize_body(x_ref, o_ref):
    x = x_ref[...]
    s = jnp.sum(x * x, axis=1, keepdims=True)
    o_ref[...] = x * lax.rsqrt(jnp.maximum(s, 1e-24))


def _normalize(x):
    return pl.pallas_call(
        _normalize_body,
        out_shape=jax.ShapeDtypeStruct((N, D), jnp.float32),
    )(x)


_sc_mesh = plsc.VectorSubcoreMesh(core_axis_name="c", subcore_axis_name="s")


@functools.partial(
    pl.kernel,
    out_type=(
        jax.ShapeDtypeStruct((NC, NPAD, D), jnp.float32),
        jax.ShapeDtypeStruct((NW, NPAD), jnp.float32),
    ),
    mesh=_sc_mesh,
    scratch_types=[
        [pltpu.VMEM((2, CHUNK), jnp.int32)] * IRING,
        [pltpu.VMEM((CHUNK, D), jnp.float32)] * NRING,
        pltpu.VMEM((NPAD,), jnp.float32),
        pltpu.VMEM_SHARED((NPAD, D), jnp.float32),
        [pltpu.SemaphoreType.DMA] * IRING,
        [pltpu.SemaphoreType.DMA] * NRING,
    ],
    compiler_params=pltpu.CompilerParams(needs_layout_passes=False),
)
def _sc_scatter(xn_hbm, idx_hbm, zrow_hbm, zdeg_hbm,
                agg_out, deg_out, ib, rows, deg_v, agg_sh, isems, gsems):
    c = lax.axis_index("c")
    s = lax.axis_index("s")
    g = c * NS + s

    pltpu.sync_copy(zrow_hbm, agg_sh.at[pl.ds(s * ROWS_PER_TILE, ROWS_PER_TILE)])
    pltpu.sync_copy(zdeg_hbm, deg_v)
    plsc.subcore_barrier()

    ones = jnp.full((16,), 1.0, dtype=jnp.float32)

    def idx_fetch(j, a):
        pltpu.async_copy(idx_hbm.at[g, j], ib[a], isems[a])

    def idx_wait(j, a):
        pltpu.make_async_copy(idx_hbm.at[g, j], ib[a], isems[a]).wait()

    def row_gather(a, b):
        pltpu.async_copy(xn_hbm.at[ib[a].at[0]], rows[b], gsems[b])

    def row_wait(a, b):
        pltpu.make_async_copy(xn_hbm.at[ib[a].at[0]], rows[b], gsems[b]).wait()

    for a in range(IRING):
        idx_fetch(a, a)
    for a in range(2):
        idx_wait(a, a)
        row_gather(a, a)

    def body(jg, carry):
        for u in range(UNROLL):
            j0 = jg * UNROLL + u
            b = u % NRING
            a = u % IRING

            row_wait(a, b)
            pltpu.sync_copy(rows[b], agg_sh.at[ib[a].at[1]], add=True)
            for i in range(CHUNK // 16):
                idx16 = ib[a][1, pl.ds(i * 16, 16)]
                plsc.addupdate_scatter(deg_v, [idx16], ones)

            @pl.when(j0 + IRING < NCHUNK)
            def _():
                idx_fetch(j0 + IRING, a)

            @pl.when(j0 + 2 < NCHUNK)
            def _():
                a2 = (u + 2) % IRING
                idx_wait(j0 + 2, a2)
                row_gather(a2, (u + 2) % NRING)

        return carry

    lax.fori_loop(0, NCHUNK // UNROLL, body, 0)
    plsc.subcore_barrier()

    pltpu.sync_copy(
        agg_sh.at[pl.ds(s * ROWS_PER_TILE, ROWS_PER_TILE)],
        agg_out.at[c, pl.ds(s * ROWS_PER_TILE, ROWS_PER_TILE)],
    )
    pltpu.sync_copy(deg_v, deg_out.at[g])


_FIN_BLK = 1024


def _final_body(a0_ref, a1_ref, deg_ref, w_ref, b_ref, noise_ref, o_ref):
    i = pl.program_id(0)
    deg = jnp.sum(deg_ref[:, pl.ds(i * _FIN_BLK, _FIN_BLK)], axis=0)
    agg = a0_ref[0] + a1_ref[0]
    mean = agg / jnp.maximum(deg, 1.0)[:, None]
    o_ref[...] = (
        jnp.dot(mean, w_ref[...], preferred_element_type=jnp.float32)
        + b_ref[...]
        + noise_ref[...] * SIGMA
    )


def _final(agg_part, deg_part, W, b2, noise):
    return pl.pallas_call(
        _final_body,
        grid=(-(-N // _FIN_BLK),),
        in_specs=[
            pl.BlockSpec((1, _FIN_BLK, D), lambda i: (0, i, 0)),
            pl.BlockSpec((1, _FIN_BLK, D), lambda i: (1, i, 0)),
            pl.BlockSpec((NW, NPAD), lambda i: (0, 0)),
            pl.BlockSpec((D, D), lambda i: (0, 0)),
            pl.BlockSpec((1, D), lambda i: (0, 0)),
            pl.BlockSpec((_FIN_BLK, D), lambda i: (i, 0)),
        ],
        out_specs=pl.BlockSpec((_FIN_BLK, D), lambda i: (i, 0)),
        out_shape=jax.ShapeDtypeStruct((N, D), jnp.float32),
    )(agg_part, agg_part, deg_part, W, b2, noise)


def kernel(x, edge_index, W, b, noise):
    xn = _normalize(x)

    src = edge_index[0]
    dst = edge_index[1]
    pad = EPAD - E
    src_p = jnp.concatenate([src, jnp.zeros((pad,), jnp.int32)])
    dst_p = jnp.concatenate([dst, jnp.full((pad,), N, jnp.int32)])
    idx4 = jnp.stack(
        [src_p.reshape(NW, NCHUNK, CHUNK), dst_p.reshape(NW, NCHUNK, CHUNK)],
        axis=2,
    )

    zrow = jnp.zeros((ROWS_PER_TILE, D), jnp.float32)
    zdeg = jnp.zeros((NPAD,), jnp.float32)

    agg_part, deg_part = _sc_scatter(xn, idx4, zrow, zdeg)

    return _final(agg_part, deg_part, W, b.reshape(1, D), noise)


# speedup vs baseline: 3.1821x; 1.1681x over previous
import functools

import jax
import jax.numpy as jnp
from jax import lax
from jax.experimental import pallas as pl
from jax.experimental.pallas import tpu as pltpu
from jax.experimental.pallas import tpu_sc as plsc

N = 10000
E = 320000
D = 128
SIGMA = 0.1

NC = 2
NS = 16
NW = NC * NS
CHUNK = 96
NRING = 3
IRING = 3
UNROLL = 3
K_C0 = 132
K_C1 = 78
TCH = NS * (K_C0 + K_C1)
EPAD = TCH * CHUNK
NPAD = 10240
ROWS_PER_TILE = NPAD // NS


def _normalize_body(x_ref, o_ref):
    x = x_ref[...]
    s = jnp.sum(x * x, axis=1, keepdims=True)
    o_ref[...] = x * lax.rsqrt(jnp.maximum(s, 1e-24))


def _normalize(x):
    return pl.pallas_call(
        _normalize_body,
        out_shape=jax.ShapeDtypeStruct((N, D), jnp.float32),
    )(x)


_sc_mesh = plsc.VectorSubcoreMesh(core_axis_name="c", subcore_axis_name="s")


@functools.partial(
    pl.kernel,
    out_type=(
        jax.ShapeDtypeStruct((NC, NPAD, D), jnp.float32),
        jax.ShapeDtypeStruct((NW, NPAD), jnp.float32),
    ),
    mesh=_sc_mesh,
    scratch_types=[
        [pltpu.VMEM((2, CHUNK), jnp.int32)] * IRING,
        [pltpu.VMEM((CHUNK, D), jnp.float32)] * NRING,
        pltpu.VMEM((NPAD,), jnp.float32),
        pltpu.VMEM_SHARED((NPAD, D), jnp.float32),
        [pltpu.SemaphoreType.DMA] * IRING,
        [pltpu.SemaphoreType.DMA] * NRING,
    ],
    compiler_params=pltpu.CompilerParams(needs_layout_passes=False),
)
def _sc_scatter(xn_hbm, idx_hbm, zrow_hbm, zdeg_hbm,
                agg_out, deg_out, ib, rows, deg_v, agg_sh, isems, gsems):
    c = lax.axis_index("c")
    s = lax.axis_index("s")
    g = c * NS + s
    start = jnp.where(c == 0, s * K_C0, NS * K_C0 + s * K_C1)
    count = jnp.where(c == 0, K_C0, K_C1)

    pltpu.sync_copy(zrow_hbm, agg_sh.at[pl.ds(s * ROWS_PER_TILE, ROWS_PER_TILE)])
    pltpu.sync_copy(zdeg_hbm, deg_v)
    plsc.subcore_barrier()

    ones = jnp.full((16,), 1.0, dtype=jnp.float32)

    def idx_fetch(j, a):
        pltpu.async_copy(idx_hbm.at[0, start + j], ib[a].at[0], isems[a])
        pltpu.async_copy(idx_hbm.at[1, start + j], ib[a].at[1], isems[a])

    def idx_wait(j, a):
        pltpu.make_async_copy(idx_hbm.at[0, start + j], ib[a].at[0], isems[a]).wait()
        pltpu.make_async_copy(idx_hbm.at[1, start + j], ib[a].at[1], isems[a]).wait()

    def row_gather(a, b):
        pltpu.async_copy(xn_hbm.at[ib[a].at[0]], rows[b], gsems[b])

    def row_wait(a, b):
        pltpu.make_async_copy(xn_hbm.at[ib[a].at[0]], rows[b], gsems[b]).wait()

    for a in range(IRING):
        idx_fetch(a, a)
    for a in range(2):
        idx_wait(a, a)
        row_gather(a, a)

    def body(jg, carry):
        for u in range(UNROLL):
            j0 = jg * UNROLL + u
            b = u % NRING
            a = u % IRING

            row_wait(a, b)
            pltpu.sync_copy(rows[b], agg_sh.at[ib[a].at[1]], add=True)
            for i in range(CHUNK // 16):
                idx16 = ib[a][1, pl.ds(i * 16, 16)]
                plsc.addupdate_scatter(deg_v, [idx16], ones)

            @pl.when(j0 + IRING < count)
            def _():
                idx_fetch(j0 + IRING, a)

            @pl.when(j0 + 2 < count)
            def _():
                a2 = (u + 2) % IRING
                idx_wait(j0 + 2, a2)
                row_gather(a2, (u + 2) % NRING)

        return carry

    lax.fori_loop(0, count // UNROLL, body, 0)
    plsc.subcore_barrier()

    pltpu.sync_copy(
        agg_sh.at[pl.ds(s * ROWS_PER_TILE, ROWS_PER_TILE)],
        agg_out.at[c, pl.ds(s * ROWS_PER_TILE, ROWS_PER_TILE)],
    )
    pltpu.sync_copy(deg_v, deg_out.at[g])


_FIN_BLK = 1024


def _final_body(a0_ref, a1_ref, deg_ref, w_ref, b_ref, noise_ref, o_ref):
    i = pl.program_id(0)
    deg = jnp.sum(deg_ref[:, pl.ds(i * _FIN_BLK, _FIN_BLK)], axis=0)
    agg = a0_ref[0] + a1_ref[0]
    mean = agg / jnp.maximum(deg, 1.0)[:, None]
    o_ref[...] = (
        jnp.dot(mean, w_ref[...], preferred_element_type=jnp.float32)
        + b_ref[...]
        + noise_ref[...] * SIGMA
    )


def _final(agg_part, deg_part, W, b2, noise):
    return pl.pallas_call(
        _final_body,
        grid=(-(-N // _FIN_BLK),),
        in_specs=[
            pl.BlockSpec((1, _FIN_BLK, D), lambda i: (0, i, 0)),
            pl.BlockSpec((1, _FIN_BLK, D), lambda i: (1, i, 0)),
            pl.BlockSpec((NW, NPAD), lambda i: (0, 0)),
            pl.BlockSpec((D, D), lambda i: (0, 0)),
            pl.BlockSpec((1, D), lambda i: (0, 0)),
            pl.BlockSpec((_FIN_BLK, D), lambda i: (i, 0)),
        ],
        out_specs=pl.BlockSpec((_FIN_BLK, D), lambda i: (i, 0)),
        out_shape=jax.ShapeDtypeStruct((N, D), jnp.float32),
    )(agg_part, agg_part, deg_part, W, b2, noise)


def kernel(x, edge_index, W, b, noise):
    xn = _normalize(x)

    pad = EPAD - E
    padcols = jnp.concatenate(
        [jnp.zeros((1, pad), jnp.int32), jnp.full((1, pad), N, jnp.int32)]
    )
    idx2 = jnp.concatenate([edge_index, padcols], axis=1).reshape(2, TCH, CHUNK)

    zrow = jnp.zeros((ROWS_PER_TILE, D), jnp.float32)
    zdeg = jnp.zeros((NPAD,), jnp.float32)

    agg_part, deg_part = _sc_scatter(xn, idx2, zrow, zdeg)

    return _final(agg_part, deg_part, W, b.reshape(1, D), noise)


# speedup vs baseline: 3.2624x; 1.0252x over previous
import functools

import jax
import jax.numpy as jnp
from jax import lax
from jax.experimental import pallas as pl
from jax.experimental.pallas import tpu as pltpu
from jax.experimental.pallas import tpu_sc as plsc

N = 10000
E = 320000
D = 128
SIGMA = 0.1

NC = 2
NS = 16
NW = NC * NS
CHUNK = 96
NRING = 3
IRING = 3
UNROLL = 3
K_C0 = 132
K_C1 = 78
TCH = NS * (K_C0 + K_C1)
EPAD = TCH * CHUNK
NPAD = 10240
ROWS_PER_TILE = NPAD // NS


def _normalize_body(x_ref, o_ref):
    x = x_ref[...]
    s = jnp.sum(x * x, axis=1, keepdims=True)
    xn = x * lax.rsqrt(jnp.maximum(s, 1e-24))
    o_ref[0] = xn
    o_ref[1] = xn


def _normalize(x):
    return pl.pallas_call(
        _normalize_body,
        grid=(-(-N // _FIN_BLK),),
        in_specs=[pl.BlockSpec((_FIN_BLK, D), lambda i: (i, 0))],
        out_specs=pl.BlockSpec((2, _FIN_BLK, D), lambda i: (0, i, 0)),
        out_shape=jax.ShapeDtypeStruct((2, N, D), jnp.float32),
    )(x)


_CORE0_COLS = NS * K_C0 * CHUNK


def _idxprep_body(e_ref, o_ref):
    col = lax.broadcasted_iota(jnp.int32, (1, E), 1)
    o_ref[0:1, :E] = e_ref[0:1, :] + jnp.where(col < _CORE0_COLS, 0, N)
    o_ref[1:2, :E] = e_ref[1:2, :]
    o_ref[0:1, E:] = jnp.full((1, EPAD - E), N, jnp.int32)
    o_ref[1:2, E:] = jnp.full((1, EPAD - E), N, jnp.int32)


def _idxprep(edge_index):
    return pl.pallas_call(
        _idxprep_body,
        in_specs=[pl.BlockSpec((2, E), lambda: (0, 0))],
        out_specs=pl.BlockSpec((2, EPAD), lambda: (0, 0)),
        out_shape=jax.ShapeDtypeStruct((2, EPAD), jnp.int32),
    )(edge_index)


_sc_mesh = plsc.VectorSubcoreMesh(core_axis_name="c", subcore_axis_name="s")


@functools.partial(
    pl.kernel,
    out_type=(
        jax.ShapeDtypeStruct((NC, NPAD, D), jnp.float32),
        jax.ShapeDtypeStruct((NW, NPAD), jnp.float32),
    ),
    mesh=_sc_mesh,
    scratch_types=[
        [pltpu.VMEM((2, CHUNK), jnp.int32)] * IRING,
        [pltpu.VMEM((CHUNK, D), jnp.float32)] * NRING,
        pltpu.VMEM((NPAD,), jnp.float32),
        pltpu.VMEM_SHARED((NPAD, D), jnp.float32),
        [pltpu.SemaphoreType.DMA] * IRING,
        [pltpu.SemaphoreType.DMA] * NRING,
    ],
    compiler_params=pltpu.CompilerParams(needs_layout_passes=False),
)
def _sc_scatter(xn_hbm, idx_hbm, zrow_hbm, zdeg_hbm,
                agg_out, deg_out, ib, rows, deg_v, agg_sh, isems, gsems):
    c = lax.axis_index("c")
    s = lax.axis_index("s")
    g = c * NS + s
    start = jnp.where(c == 0, s * K_C0, NS * K_C0 + s * K_C1)
    count = jnp.where(c == 0, K_C0, K_C1)

    pltpu.sync_copy(zrow_hbm, agg_sh.at[pl.ds(s * ROWS_PER_TILE, ROWS_PER_TILE)])
    pltpu.sync_copy(zdeg_hbm, deg_v)
    plsc.subcore_barrier()

    ones = jnp.full((16,), 1.0, dtype=jnp.float32)

    def idx_fetch(j, a):
        pltpu.async_copy(idx_hbm.at[0, start + j], ib[a].at[0], isems[a])
        pltpu.async_copy(idx_hbm.at[1, start + j], ib[a].at[1], isems[a])

    def idx_wait(j, a):
        pltpu.make_async_copy(idx_hbm.at[0, start + j], ib[a].at[0], isems[a]).wait()
        pltpu.make_async_copy(idx_hbm.at[1, start + j], ib[a].at[1], isems[a]).wait()

    def row_gather(a, b):
        pltpu.async_copy(xn_hbm.at[ib[a].at[0]], rows[b], gsems[b])

    def row_wait(a, b):
        pltpu.make_async_copy(xn_hbm.at[ib[a].at[0]], rows[b], gsems[b]).wait()

    for a in range(IRING):
        idx_fetch(a, a)
    for a in range(2):
        idx_wait(a, a)
        row_gather(a, a)

    def body(jg, carry):
        for u in range(UNROLL):
            j0 = jg * UNROLL + u
            b = u % NRING
            a = u % IRING

            row_wait(a, b)
            pltpu.sync_copy(rows[b], agg_sh.at[ib[a].at[1]], add=True)
            for i in range(CHUNK // 16):
                idx16 = ib[a][1, pl.ds(i * 16, 16)]
                plsc.addupdate_scatter(deg_v, [idx16], ones)

            @pl.when(j0 + IRING < count)
            def _():
                idx_fetch(j0 + IRING, a)

            @pl.when(j0 + 2 < count)
            def _():
                a2 = (u + 2) % IRING
                idx_wait(j0 + 2, a2)
                row_gather(a2, (u + 2) % NRING)

        return carry

    lax.fori_loop(0, count // UNROLL, body, 0)
    plsc.subcore_barrier()

    pltpu.sync_copy(
        agg_sh.at[pl.ds(s * ROWS_PER_TILE, ROWS_PER_TILE)],
        agg_out.at[c, pl.ds(s * ROWS_PER_TILE, ROWS_PER_TILE)],
    )
    pltpu.sync_copy(deg_v, deg_out.at[g])


_FIN_BLK = 1024


def _final_body(a0_ref, a1_ref, deg_ref, w_ref, b_ref, noise_ref, o_ref):
    i = pl.program_id(0)
    deg = jnp.sum(deg_ref[:, pl.ds(i * _FIN_BLK, _FIN_BLK)], axis=0)
    agg = a0_ref[0] + a1_ref[0]
    mean = agg / jnp.maximum(deg, 1.0)[:, None]
    o_ref[...] = (
        jnp.dot(mean, w_ref[...], preferred_element_type=jnp.float32)
        + b_ref[...]
        + noise_ref[...] * SIGMA
    )


def _final(agg_part, deg_part, W, b2, noise):
    return pl.pallas_call(
        _final_body,
        grid=(-(-N // _FIN_BLK),),
        in_specs=[
            pl.BlockSpec((1, _FIN_BLK, D), lambda i: (0, i, 0)),
            pl.BlockSpec((1, _FIN_BLK, D), lambda i: (1, i, 0)),
            pl.BlockSpec((NW, NPAD), lambda i: (0, 0)),
            pl.BlockSpec((D, D), lambda i: (0, 0)),
            pl.BlockSpec((1, D), lambda i: (0, 0)),
            pl.BlockSpec((_FIN_BLK, D), lambda i: (i, 0)),
        ],
        out_specs=pl.BlockSpec((_FIN_BLK, D), lambda i: (i, 0)),
        out_shape=jax.ShapeDtypeStruct((N, D), jnp.float32),
    )(agg_part, agg_part, deg_part, W, b2, noise)


def kernel(x, edge_index, W, b, noise):
    xn2 = _normalize(x).reshape(2 * N, D)
    idx2 = _idxprep(edge_index).reshape(2, TCH, CHUNK)

    zrow = jnp.zeros((ROWS_PER_TILE, D), jnp.float32)
    zdeg = jnp.zeros((NPAD,), jnp.float32)

    agg_part, deg_part = _sc_scatter(xn2, idx2, zrow, zdeg)

    return _final(agg_part, deg_part, W, b.reshape(1, D), noise)
